# baseline (speedup 1.0000x reference)
import functools

import jax
import jax.numpy as jnp
from jax import lax
from jax.experimental import pallas as pl
from jax.experimental.pallas import tpu as pltpu, tpu_sc as plsc

ROWS = 4096
COLS = 26
VOCAB = 1000
PLANE = COLS * VOCAB
TOTAL = ROWS * PLANE
NUM_WORKERS = 32
PLANES_PER_WORKER = ROWS // NUM_WORKERS
POS_PER_WORKER = PLANES_PER_WORKER * COLS
L = 16
K = 128
CHUNKS = POS_PER_WORKER // K
MEMSET_BLK = 1_024_000


def _memset_body(o_ref):
    o_ref[...] = jnp.zeros((MEMSET_BLK,), jnp.float32)


_memset = pl.pallas_call(
    _memset_body,
    grid=(TOTAL // MEMSET_BLK,),
    out_specs=pl.BlockSpec((MEMSET_BLK,), lambda i: (i,)),
    out_shape=jax.ShapeDtypeStruct((TOTAL,), jnp.float32),
)


def _scatter_body(x_hbm, out_ref, xs_v, idx0, idx1, ones_v, sem0, sem1):
    wid = lax.axis_index("c") * 16 + lax.axis_index("s")
    base = wid * PLANES_PER_WORKER

    pltpu.sync_copy(x_hbm.at[pl.ds(base, PLANES_PER_WORKER)], xs_v)

    iota = lax.iota(jnp.int32, L)
    for m in range(K // L):
        ones_v[pl.ds(m * L, L)] = jnp.full((L,), 1.0, jnp.float32)

    idxs = (idx0, idx1)
    sems = (sem0, sem1)

    def fill_idx(c, idx_ref):
        for m in range(K // L):
            r = c * K + m * L + iota
            poff = r // COLS
            j = r - poff * COLS
            cols = plsc.load_gather(xs_v, [poff, j])
            idx_ref[pl.ds(m * L, L)] = (base + poff) * PLANE + j * VOCAB + cols

    def fire(b):
        pltpu.async_copy(ones_v, out_ref.at[idxs[b]], sems[b])

    for b in range(2):
        fill_idx(b, idxs[b])
        fire(b)

    def step(g, carry):
        for b in range(2):
            pltpu.make_async_copy(ones_v, out_ref.at[idxs[b]], sems[b]).wait()
            fill_idx(2 + 2 * g + b, idxs[b])
            fire(b)
        return carry

    lax.fori_loop(0, (CHUNKS - 2) // 2, step, 0)

    pltpu.make_async_copy(ones_v, out_ref.at[idx0], sem0).wait()
    pltpu.make_async_copy(ones_v, out_ref.at[idx1], sem1).wait()


_scatter = functools.partial(
    pl.kernel,
    mesh=plsc.VectorSubcoreMesh(core_axis_name="c", subcore_axis_name="s"),
    compiler_params=pltpu.CompilerParams(
        use_tc_tiling_on_sc=False, needs_layout_passes=False),
    scratch_types=[
        pltpu.VMEM((PLANES_PER_WORKER, COLS), jnp.int32),
        pltpu.VMEM((K,), jnp.int32),
        pltpu.VMEM((K,), jnp.int32),
        pltpu.VMEM((K,), jnp.float32),
        pltpu.SemaphoreType.DMA,
        pltpu.SemaphoreType.DMA,
    ],
)(_scatter_body)


def kernel(x):
    out = jax.new_ref(_memset())
    _scatter(x, out)
    return out[...].reshape(ROWS, COLS, VOCAB)
